# baseline (speedup 1.0000x reference)
import jax
import jax.numpy as jnp
from jax.experimental import pallas as pl
from jax.experimental.pallas import tpu as pltpu

_B = 8
_N = 20000
_F = 85
_C = 80
_THRES = 0.05

_PG = 17
_NG = _F // _PG
_CH = 128

_CHUNKS = [(j, _CH) for j in range(0, _N - _N % _CH, _CH)]
if _N % _CH:
    _CHUNKS.append((_N - _N % _CH, _N % _CH))


def _body(in_ref, det_ref, mask_ref, best_ref, bidx_ref):
    g = pl.program_id(0)

    @pl.when(g == 0)
    def _init():
        for j, w in _CHUNKS:
            sl = pl.ds(j, w)
            obj = in_ref[4, :, sl]
            mask = obj >= _THRES
            zero = jnp.zeros_like(obj)
            for c in range(4):
                det_ref[c, :, sl] = jnp.where(mask, in_ref[c, :, sl], zero)
            det_ref[4, :, sl] = jnp.where(mask, obj, zero)
            mask_ref[:, sl] = mask.astype(jnp.int8)
            best = in_ref[5, :, sl]
            bidx = jnp.zeros_like(best)
            for c in range(1, _PG - 5):
                v = in_ref[5 + c, :, sl]
                upd = v > best
                bidx = jnp.where(upd, jnp.float32(c), bidx)
                best = jnp.maximum(best, v)
            best_ref[:, sl] = best
            bidx_ref[:, sl] = bidx

    @pl.when(g > 0)
    def _scan():
        base = (g * _PG - 5).astype(jnp.float32)
        for j, w in _CHUNKS:
            sl = pl.ds(j, w)
            best = best_ref[:, sl]
            bidx = bidx_ref[:, sl]
            for i in range(_PG):
                v = in_ref[i, :, sl]
                upd = v > best
                bidx = jnp.where(upd, base + i, bidx)
                best = jnp.maximum(best, v)
            best_ref[:, sl] = best
            bidx_ref[:, sl] = bidx

    @pl.when(g == _NG - 1)
    def _final():
        for j, w in _CHUNKS:
            sl = pl.ds(j, w)
            mask = det_ref[4, :, sl] >= _THRES
            zero = jnp.zeros_like(best_ref[:, sl])
            det_ref[5, :, sl] = jnp.where(mask, best_ref[:, sl], zero)
            det_ref[6, :, sl] = jnp.where(mask, bidx_ref[:, sl], zero)


def kernel(prediction):
    xp = jnp.transpose(prediction, (2, 0, 1))
    det_p, mask = pl.pallas_call(
        _body,
        grid=(_NG,),
        in_specs=[pl.BlockSpec((_PG, _B, _N), lambda g: (g, 0, 0))],
        out_specs=[
            pl.BlockSpec((7, _B, _N), lambda g: (0, 0, 0)),
            pl.BlockSpec((_B, _N), lambda g: (0, 0)),
        ],
        out_shape=[
            jax.ShapeDtypeStruct((7, _B, _N), jnp.float32),
            jax.ShapeDtypeStruct((_B, _N), jnp.int8),
        ],
        scratch_shapes=[
            pltpu.VMEM((_B, _N), jnp.float32),
            pltpu.VMEM((_B, _N), jnp.float32),
        ],
    )(xp)
    return jnp.transpose(det_p, (1, 2, 0)), mask.astype(jnp.bool_)


# speedup vs baseline: 1.0345x; 1.0345x over previous
import jax
import jax.numpy as jnp
from jax.experimental import pallas as pl

_B = 8
_N = 20000
_F = 85
_C = 80
_THRES = 0.05

_BLKN = 3584
_CHUNK = 128
_NB = -(-_N // _BLKN)


def _postprocess_block(in_ref, det_ref, mask_ref):
    for j in range(0, _BLKN, _CHUNK):
        sl = pl.ds(j, _CHUNK)
        obj = in_ref[4, :, sl]
        mask = obj >= _THRES
        best = in_ref[5, :, sl]
        bidx = jnp.zeros_like(best)
        for c in range(1, _C):
            v = in_ref[5 + c, :, sl]
            upd = v > best
            bidx = jnp.where(upd, jnp.float32(c), bidx)
            best = jnp.maximum(best, v)
        zero = jnp.zeros_like(best)
        for c in range(4):
            det_ref[c, :, sl] = jnp.where(mask, in_ref[c, :, sl], zero)
        det_ref[4, :, sl] = jnp.where(mask, obj, zero)
        det_ref[5, :, sl] = jnp.where(mask, best, zero)
        det_ref[6, :, sl] = jnp.where(mask, bidx, zero)
        mask_ref[:, sl] = mask.astype(jnp.int8)


def kernel(prediction):
    xp = jnp.transpose(prediction, (2, 0, 1))
    det_p, mask = pl.pallas_call(
        _postprocess_block,
        grid=(_NB,),
        in_specs=[pl.BlockSpec((_F, _B, _BLKN), lambda i: (0, 0, i))],
        out_specs=[
            pl.BlockSpec((7, _B, _BLKN), lambda i: (0, 0, i)),
            pl.BlockSpec((_B, _BLKN), lambda i: (0, i)),
        ],
        out_shape=[
            jax.ShapeDtypeStruct((7, _B, _N), jnp.float32),
            jax.ShapeDtypeStruct((_B, _N), jnp.int8),
        ],
    )(xp)
    return jnp.transpose(det_p, (1, 2, 0)), mask.astype(jnp.bool_)


# speedup vs baseline: 1.0505x; 1.0154x over previous
import jax
import jax.numpy as jnp
from jax.experimental import pallas as pl

_B = 8
_N = 20000
_F = 85
_C = 80
_THRES = 0.05

_BLKN = 3072
_CHUNK = 128
_NB = -(-_N // _BLKN)


def _postprocess_block(in_ref, det_ref, mask_ref):
    for j in range(0, _BLKN, _CHUNK):
        sl = pl.ds(j, _CHUNK)
        obj = in_ref[4, :, sl]
        mask = obj >= _THRES
        best = in_ref[5, :, sl]
        bidx = jnp.zeros_like(best)
        for c in range(1, _C):
            v = in_ref[5 + c, :, sl]
            upd = v > best
            bidx = jnp.where(upd, jnp.float32(c), bidx)
            best = jnp.maximum(best, v)
        zero = jnp.zeros_like(best)
        for c in range(4):
            det_ref[c, :, sl] = jnp.where(mask, in_ref[c, :, sl], zero)
        det_ref[4, :, sl] = jnp.where(mask, obj, zero)
        det_ref[5, :, sl] = jnp.where(mask, best, zero)
        det_ref[6, :, sl] = jnp.where(mask, bidx, zero)
        mask_ref[:, sl] = mask.astype(jnp.int8)


def kernel(prediction):
    xp = jnp.transpose(prediction, (2, 0, 1))
    det_p, mask = pl.pallas_call(
        _postprocess_block,
        grid=(_NB,),
        in_specs=[pl.BlockSpec((_F, _B, _BLKN), lambda i: (0, 0, i))],
        out_specs=[
            pl.BlockSpec((7, _B, _BLKN), lambda i: (0, 0, i)),
            pl.BlockSpec((_B, _BLKN), lambda i: (0, i)),
        ],
        out_shape=[
            jax.ShapeDtypeStruct((7, _B, _N), jnp.float32),
            jax.ShapeDtypeStruct((_B, _N), jnp.int8),
        ],
    )(xp)
    return jnp.transpose(det_p, (1, 2, 0)), mask.astype(jnp.bool_)
